# Initial kernel scaffold; baseline (speedup 1.0000x reference)
import jax
import jax.numpy as jnp
from jax.experimental import pallas as pl


def kernel(x, edge_index, W0, b0, Wg, bg, Wggc, W_ih, W_hh, b_ih, b_hh):
    raise NotImplementedError("write your pallas kernel here")


# speedup vs baseline: 1.0265x; 1.0265x over previous
import functools

import jax
import jax.numpy as jnp
from jax import lax
from jax.experimental import pallas as pl
from jax.experimental.pallas import tpu as pltpu
from jax.experimental.pallas import tpu_sc as plsc

N = 10000
E = 320000
D = 128
NUM_GCN = 30
NUM_GGC = 8

NC = 2
NS = 16
NW = NC * NS
CH = 128
EPW = -(-E // (NW * CH)) * CH
CHUNKS = EPW // CH
EPAD = EPW * NW

NPAD = 10016
RPS = NPAD // NS


def _make_agg(width):
    mesh = plsc.VectorSubcoreMesh(
        core_axis_name="c", subcore_axis_name="s", num_cores=NC, num_subcores=NS
    )

    @functools.partial(
        pl.kernel,
        out_type=jax.ShapeDtypeStruct((NC, NS, RPS, width), jnp.float32),
        mesh=mesh,
        scratch_types=[
            pltpu.VMEM((CHUNKS, CH), jnp.int32),
            pltpu.VMEM((CHUNKS, CH), jnp.int32),
            pltpu.VMEM((CH, width), jnp.float32),
            pltpu.VMEM_SHARED((NPAD, width), jnp.float32),
        ],
    )
    def agg(t_hbm, src_hbm, dst_hbm, z_hbm, out_hbm, src_v, dst_v, rows_v, acc):
        c = lax.axis_index("c")
        s = lax.axis_index("s")
        wid = c * NS + s
        pltpu.sync_copy(z_hbm, acc.at[pl.ds(s * RPS, RPS)])
        pltpu.sync_copy(src_hbm.at[wid], src_v)
        pltpu.sync_copy(dst_hbm.at[wid], dst_v)
        plsc.subcore_barrier()

        def body(j, carry):
            pltpu.sync_copy(t_hbm.at[src_v.at[j]], rows_v)
            pltpu.sync_copy(rows_v, acc.at[dst_v.at[j]], add=True)
            return carry

        lax.fori_loop(0, CHUNKS, body, 0)
        plsc.subcore_barrier()
        pltpu.sync_copy(acc.at[pl.ds(s * RPS, RPS)], out_hbm.at[c, s])

    return agg


_agg_d = _make_agg(D)


BLK = 1000
GRID = N // BLK


def _tc_pre_body(x_r, w0_r, b0_r, wg0_r, c0_r, c1_r, dinv_o, t_o):
    deg = c0_r[:, 0:1] + c1_r[:, 0:1] + 1.0
    dinv = lax.rsqrt(deg)
    h = jnp.maximum(
        jnp.dot(x_r[...], w0_r[...], preferred_element_type=jnp.float32)
        + b0_r[...], 0.0)
    dinv_o[...] = jnp.broadcast_to(dinv, (BLK, D))
    t_o[...] = jnp.dot(h, wg0_r[...], preferred_element_type=jnp.float32)


def _tc_pre(x, w0, b0, wg0, c0, c1):
    return pl.pallas_call(
        _tc_pre_body,
        grid=(GRID,),
        in_specs=[
            pl.BlockSpec((BLK, 2 * D), lambda i: (i, 0)),
            pl.BlockSpec((2 * D, D), lambda i: (0, 0)),
            pl.BlockSpec((1, D), lambda i: (0, 0)),
            pl.BlockSpec((D, D), lambda i: (0, 0)),
            pl.BlockSpec((BLK, 16), lambda i: (i, 0)),
            pl.BlockSpec((BLK, 16), lambda i: (i, 0)),
        ],
        out_specs=[
            pl.BlockSpec((BLK, D), lambda i: (i, 0)),
            pl.BlockSpec((BLK, D), lambda i: (i, 0)),
        ],
        out_shape=[
            jax.ShapeDtypeStruct((N, D), jnp.float32),
            jax.ShapeDtypeStruct((N, D), jnp.float32),
        ],
    )(x, w0, b0, wg0, c0, c1)


def _tc_mm_body(m_r, bg_r, w_r, t_o):
    h = m_r[...] + bg_r[...]
    t_o[...] = jnp.dot(h, w_r[...], preferred_element_type=jnp.float32)


def _tc_mm(m, bg, w):
    return pl.pallas_call(
        _tc_mm_body,
        grid=(GRID,),
        in_specs=[
            pl.BlockSpec((BLK, D), lambda i: (i, 0)),
            pl.BlockSpec((1, D), lambda i: (0, 0)),
            pl.BlockSpec((D, D), lambda i: (0, 0)),
        ],
        out_specs=pl.BlockSpec((BLK, D), lambda i: (i, 0)),
        out_shape=jax.ShapeDtypeStruct((N, D), jnp.float32),
    )(m, bg, w)


def _tc_entry_body(m_r, bg_r, w_r, h_o, msg_o):
    h = m_r[...] + bg_r[...]
    h_o[...] = h
    msg_o[...] = jnp.dot(h, w_r[...], preferred_element_type=jnp.float32)


def _tc_entry(m, bg, w):
    return pl.pallas_call(
        _tc_entry_body,
        grid=(GRID,),
        in_specs=[
            pl.BlockSpec((BLK, D), lambda i: (i, 0)),
            pl.BlockSpec((1, D), lambda i: (0, 0)),
            pl.BlockSpec((D, D), lambda i: (0, 0)),
        ],
        out_specs=[
            pl.BlockSpec((BLK, D), lambda i: (i, 0)),
            pl.BlockSpec((BLK, D), lambda i: (i, 0)),
        ],
        out_shape=[
            jax.ShapeDtypeStruct((N, D), jnp.float32),
            jax.ShapeDtypeStruct((N, D), jnp.float32),
        ],
    )(m, bg, w)


def _tc_gru_body(m_r, h_r, wih_r, whh_r, bih_r, bhh_r, wn_r, h_o, msg_o):
    gi = jnp.dot(m_r[...], wih_r[...], preferred_element_type=jnp.float32) + bih_r[...]
    gh = jnp.dot(h_r[...], whh_r[...], preferred_element_type=jnp.float32) + bhh_r[...]
    h = h_r[...]
    r = jax.nn.sigmoid(gi[:, 0:D] + gh[:, 0:D])
    z = jax.nn.sigmoid(gi[:, D:2 * D] + gh[:, D:2 * D])
    n = jnp.tanh(gi[:, 2 * D:3 * D] + r * gh[:, 2 * D:3 * D])
    h2 = (1.0 - z) * n + z * h
    h_o[...] = h2
    msg_o[...] = jnp.dot(h2, wn_r[...], preferred_element_type=jnp.float32)


def _tc_gru(m, h, wih, whh, bih, bhh, wn):
    return pl.pallas_call(
        _tc_gru_body,
        grid=(GRID,),
        in_specs=[
            pl.BlockSpec((BLK, D), lambda i: (i, 0)),
            pl.BlockSpec((BLK, D), lambda i: (i, 0)),
            pl.BlockSpec((D, 3 * D), lambda i: (0, 0)),
            pl.BlockSpec((D, 3 * D), lambda i: (0, 0)),
            pl.BlockSpec((1, 3 * D), lambda i: (0, 0)),
            pl.BlockSpec((1, 3 * D), lambda i: (0, 0)),
            pl.BlockSpec((D, D), lambda i: (0, 0)),
        ],
        out_specs=[
            pl.BlockSpec((BLK, D), lambda i: (i, 0)),
            pl.BlockSpec((BLK, D), lambda i: (i, 0)),
        ],
        out_shape=[
            jax.ShapeDtypeStruct((N, D), jnp.float32),
            jax.ShapeDtypeStruct((N, D), jnp.float32),
        ],
    )(m, h, wih, whh, bih, bhh, wn)


def kernel(x, edge_index, W0, b0, Wg, bg, Wggc, W_ih, W_hh, b_ih, b_hh):
    src = edge_index[0]
    dst = edge_index[1]
    padn = EPAD - E
    srcp = jnp.concatenate([src, jnp.zeros((padn,), jnp.int32)])
    dstp = jnp.concatenate([dst, jnp.full((padn,), N, jnp.int32)])
    srcR = srcp.reshape(NW, CHUNKS, CH)
    dstR = dstp.reshape(NW, CHUNKS, CH)

    zD = jnp.zeros((RPS, D), jnp.float32)
    onesD = jnp.ones((N, D), jnp.float32)

    cnt = _agg_d(onesD, srcR, dstR, zD).reshape(NC, NPAD, D)
    c0 = cnt[0, :N, :16]
    c1 = cnt[1, :N, :16]

    b0r = b0.reshape(1, D)
    dinv_b, t = _tc_pre(x, W0, b0r, Wg[0], c0, c1)
    dinv = dinv_b[:, 0]

    loop = jnp.arange(N, dtype=src.dtype)
    src_a = jnp.concatenate([src, loop])
    dst_a = jnp.concatenate([dst, loop])
    norm = dinv[src_a] * dinv[dst_a]

    def agg_gcn(t):
        return jax.ops.segment_sum(t[src_a] * norm[:, None], dst_a,
                                   num_segments=N)

    for i in range(NUM_GCN - 1):
        m = agg_gcn(t)
        t = _tc_mm(m, bg[i].reshape(1, D), Wg[i + 1])

    m = agg_gcn(t)
    h, msg = _tc_entry(m, bg[NUM_GCN - 1].reshape(1, D), Wggc[0])

    wih = W_ih.T
    whh = W_hh.T
    bihr = b_ih.reshape(1, 3 * D)
    bhhr = b_hh.reshape(1, 3 * D)
    for l in range(NUM_GGC):
        m = jax.ops.segment_sum(msg[src], dst, num_segments=N)
        wn = Wggc[(l + 1) % NUM_GGC]
        h, msg = _tc_gru(m, h, wih, whh, bihr, bhhr, wn)

    return h


# speedup vs baseline: 1.2318x; 1.2000x over previous
import functools

import jax
import jax.numpy as jnp
from jax import lax
from jax.experimental import pallas as pl
from jax.experimental.pallas import tpu as pltpu
from jax.experimental.pallas import tpu_sc as plsc

N = 10000
E = 320000
D = 128
NUM_GCN = 30
NUM_GGC = 8

NC = 2
NS = 16
NW = NC * NS
CH = 128
EPW = -(-E // (NW * CH)) * CH
CHUNKS = EPW // CH
EPAD = EPW * NW

NPAD = 10016
RPS = NPAD // NS


def _make_agg(width):
    mesh = plsc.VectorSubcoreMesh(
        core_axis_name="c", subcore_axis_name="s", num_cores=NC, num_subcores=NS
    )

    @functools.partial(
        pl.kernel,
        out_type=jax.ShapeDtypeStruct((NC, NS, RPS, width), jnp.float32),
        mesh=mesh,
        scratch_types=[
            pltpu.VMEM((CHUNKS, CH), jnp.int32),
            pltpu.VMEM((CHUNKS, CH), jnp.int32),
            pltpu.VMEM((CH, width), jnp.float32),
            pltpu.VMEM_SHARED((NPAD, width), jnp.float32),
        ],
    )
    def agg(t_hbm, src_hbm, dst_hbm, z_hbm, out_hbm, src_v, dst_v, rows_v, acc):
        c = lax.axis_index("c")
        s = lax.axis_index("s")
        wid = c * NS + s
        pltpu.sync_copy(z_hbm, acc.at[pl.ds(s * RPS, RPS)])
        pltpu.sync_copy(src_hbm.at[wid], src_v)
        pltpu.sync_copy(dst_hbm.at[wid], dst_v)
        plsc.subcore_barrier()

        def body(j, carry):
            pltpu.sync_copy(t_hbm.at[src_v.at[j]], rows_v)
            pltpu.sync_copy(rows_v, acc.at[dst_v.at[j]], add=True)
            return carry

        lax.fori_loop(0, CHUNKS, body, 0)
        plsc.subcore_barrier()
        pltpu.sync_copy(acc.at[pl.ds(s * RPS, RPS)], out_hbm.at[c, s])

    return agg


_agg_d = _make_agg(D)


def _make_gather(chunks):
    mesh = plsc.VectorSubcoreMesh(
        core_axis_name="c", subcore_axis_name="s", num_cores=NC, num_subcores=NS
    )

    @functools.partial(
        pl.kernel,
        out_type=jax.ShapeDtypeStruct((NW, chunks * CH, D), jnp.float32),
        mesh=mesh,
        scratch_types=[
            pltpu.VMEM((chunks, CH), jnp.int32),
            pltpu.VMEM((2, CH, D), jnp.float32),
            pltpu.SemaphoreType.DMA((2,)),
            pltpu.SemaphoreType.DMA((2,)),
        ],
    )
    def gat(t_hbm, idx_hbm, out_hbm, idx_v, ring, gsem, wsem):
        c = lax.axis_index("c")
        s = lax.axis_index("s")
        wid = c * NS + s
        pltpu.sync_copy(idx_hbm.at[wid], idx_v)
        pltpu.async_copy(t_hbm.at[idx_v.at[0]], ring.at[0], gsem.at[0])

        def body(j, carry):
            par = lax.rem(j, 2)
            nxt = lax.rem(j + 1, 2)

            @pl.when(j + 1 < chunks)
            def _():
                @pl.when(j >= 1)
                def _():
                    pltpu.make_async_copy(
                        ring.at[nxt],
                        out_hbm.at[wid, pl.ds((j - 1) * CH, CH)],
                        wsem.at[nxt],
                    ).wait()

                pltpu.async_copy(
                    t_hbm.at[idx_v.at[j + 1]], ring.at[nxt], gsem.at[nxt])

            pltpu.make_async_copy(
                t_hbm.at[idx_v.at[j]], ring.at[par], gsem.at[par]).wait()
            pltpu.async_copy(
                ring.at[par], out_hbm.at[wid, pl.ds(j * CH, CH)], wsem.at[par])
            return carry

        lax.fori_loop(0, chunks, body, 0)
        last = lax.rem(chunks - 1, 2)

        @pl.when(chunks >= 2)
        def _():
            pltpu.make_async_copy(
                ring.at[1 - last],
                out_hbm.at[wid, pl.ds((chunks - 2) * CH, CH)],
                wsem.at[1 - last],
            ).wait()

        pltpu.make_async_copy(
            ring.at[last],
            out_hbm.at[wid, pl.ds((chunks - 1) * CH, CH)],
            wsem.at[last],
        ).wait()

    return gat


EA = E + N
EPW_A = -(-EA // (NW * CH)) * CH
CHUNKS_A = EPW_A // CH
EPAD_A = EPW_A * NW

_gather_e = _make_gather(CHUNKS)
_gather_a = _make_gather(CHUNKS_A)


BLK = 1000
GRID = N // BLK


def _tc_pre_body(x_r, w0_r, b0_r, wg0_r, c0_r, c1_r, dinv_o, t_o):
    deg = c0_r[:, 0:1] + c1_r[:, 0:1] + 1.0
    dinv = lax.rsqrt(deg)
    h = jnp.maximum(
        jnp.dot(x_r[...], w0_r[...], preferred_element_type=jnp.float32)
        + b0_r[...], 0.0)
    dinv_o[...] = jnp.broadcast_to(dinv, (BLK, D))
    t_o[...] = jnp.dot(h, wg0_r[...], preferred_element_type=jnp.float32)


def _tc_pre(x, w0, b0, wg0, c0, c1):
    return pl.pallas_call(
        _tc_pre_body,
        grid=(GRID,),
        in_specs=[
            pl.BlockSpec((BLK, 2 * D), lambda i: (i, 0)),
            pl.BlockSpec((2 * D, D), lambda i: (0, 0)),
            pl.BlockSpec((1, D), lambda i: (0, 0)),
            pl.BlockSpec((D, D), lambda i: (0, 0)),
            pl.BlockSpec((BLK, 16), lambda i: (i, 0)),
            pl.BlockSpec((BLK, 16), lambda i: (i, 0)),
        ],
        out_specs=[
            pl.BlockSpec((BLK, D), lambda i: (i, 0)),
            pl.BlockSpec((BLK, D), lambda i: (i, 0)),
        ],
        out_shape=[
            jax.ShapeDtypeStruct((N, D), jnp.float32),
            jax.ShapeDtypeStruct((N, D), jnp.float32),
        ],
    )(x, w0, b0, wg0, c0, c1)


def _tc_mm_body(m_r, bg_r, w_r, t_o):
    h = m_r[...] + bg_r[...]
    t_o[...] = jnp.dot(h, w_r[...], preferred_element_type=jnp.float32)


def _tc_mm(m, bg, w):
    return pl.pallas_call(
        _tc_mm_body,
        grid=(GRID,),
        in_specs=[
            pl.BlockSpec((BLK, D), lambda i: (i, 0)),
            pl.BlockSpec((1, D), lambda i: (0, 0)),
            pl.BlockSpec((D, D), lambda i: (0, 0)),
        ],
        out_specs=pl.BlockSpec((BLK, D), lambda i: (i, 0)),
        out_shape=jax.ShapeDtypeStruct((N, D), jnp.float32),
    )(m, bg, w)


def _tc_entry_body(m_r, bg_r, w_r, h_o, msg_o):
    h = m_r[...] + bg_r[...]
    h_o[...] = h
    msg_o[...] = jnp.dot(h, w_r[...], preferred_element_type=jnp.float32)


def _tc_entry(m, bg, w):
    return pl.pallas_call(
        _tc_entry_body,
        grid=(GRID,),
        in_specs=[
            pl.BlockSpec((BLK, D), lambda i: (i, 0)),
            pl.BlockSpec((1, D), lambda i: (0, 0)),
            pl.BlockSpec((D, D), lambda i: (0, 0)),
        ],
        out_specs=[
            pl.BlockSpec((BLK, D), lambda i: (i, 0)),
            pl.BlockSpec((BLK, D), lambda i: (i, 0)),
        ],
        out_shape=[
            jax.ShapeDtypeStruct((N, D), jnp.float32),
            jax.ShapeDtypeStruct((N, D), jnp.float32),
        ],
    )(m, bg, w)


def _tc_gru_body(m_r, h_r, wih_r, whh_r, bih_r, bhh_r, wn_r, h_o, msg_o):
    gi = jnp.dot(m_r[...], wih_r[...], preferred_element_type=jnp.float32) + bih_r[...]
    gh = jnp.dot(h_r[...], whh_r[...], preferred_element_type=jnp.float32) + bhh_r[...]
    h = h_r[...]
    r = jax.nn.sigmoid(gi[:, 0:D] + gh[:, 0:D])
    z = jax.nn.sigmoid(gi[:, D:2 * D] + gh[:, D:2 * D])
    n = jnp.tanh(gi[:, 2 * D:3 * D] + r * gh[:, 2 * D:3 * D])
    h2 = (1.0 - z) * n + z * h
    h_o[...] = h2
    msg_o[...] = jnp.dot(h2, wn_r[...], preferred_element_type=jnp.float32)


def _tc_gru(m, h, wih, whh, bih, bhh, wn):
    return pl.pallas_call(
        _tc_gru_body,
        grid=(GRID,),
        in_specs=[
            pl.BlockSpec((BLK, D), lambda i: (i, 0)),
            pl.BlockSpec((BLK, D), lambda i: (i, 0)),
            pl.BlockSpec((D, 3 * D), lambda i: (0, 0)),
            pl.BlockSpec((D, 3 * D), lambda i: (0, 0)),
            pl.BlockSpec((1, 3 * D), lambda i: (0, 0)),
            pl.BlockSpec((1, 3 * D), lambda i: (0, 0)),
            pl.BlockSpec((D, D), lambda i: (0, 0)),
        ],
        out_specs=[
            pl.BlockSpec((BLK, D), lambda i: (i, 0)),
            pl.BlockSpec((BLK, D), lambda i: (i, 0)),
        ],
        out_shape=[
            jax.ShapeDtypeStruct((N, D), jnp.float32),
            jax.ShapeDtypeStruct((N, D), jnp.float32),
        ],
    )(m, h, wih, whh, bih, bhh, wn)


def kernel(x, edge_index, W0, b0, Wg, bg, Wggc, W_ih, W_hh, b_ih, b_hh):
    src = edge_index[0]
    dst = edge_index[1]
    padn = EPAD - E
    srcp = jnp.concatenate([src, jnp.zeros((padn,), jnp.int32)])
    dstp = jnp.concatenate([dst, jnp.full((padn,), N, jnp.int32)])
    srcR = srcp.reshape(NW, CHUNKS, CH)
    dstR = dstp.reshape(NW, CHUNKS, CH)

    zD = jnp.zeros((RPS, D), jnp.float32)
    onesD = jnp.ones((N, D), jnp.float32)

    cnt = _agg_d(onesD, srcR, dstR, zD).reshape(NC, NPAD, D)
    c0 = cnt[0, :N, :16]
    c1 = cnt[1, :N, :16]

    b0r = b0.reshape(1, D)
    dinv_b, t = _tc_pre(x, W0, b0r, Wg[0], c0, c1)
    dinv = dinv_b[:, 0]

    loop = jnp.arange(N, dtype=src.dtype)
    src_a = jnp.concatenate([src, loop])
    dst_a = jnp.concatenate([dst, loop])
    norm = dinv[src_a] * dinv[dst_a]

    pad_a = EPAD_A - EA
    srcaR = jnp.concatenate([src_a, jnp.zeros((pad_a,), jnp.int32)]).reshape(
        NW, CHUNKS_A, CH)

    def agg_gcn(t):
        g = _gather_a(t, srcaR).reshape(EPAD_A, D)[:EA]
        return jax.ops.segment_sum(g * norm[:, None], dst_a, num_segments=N)

    for i in range(NUM_GCN - 1):
        m = agg_gcn(t)
        t = _tc_mm(m, bg[i].reshape(1, D), Wg[i + 1])

    m = agg_gcn(t)
    h, msg = _tc_entry(m, bg[NUM_GCN - 1].reshape(1, D), Wggc[0])

    wih = W_ih.T
    whh = W_hh.T
    bihr = b_ih.reshape(1, 3 * D)
    bhhr = b_hh.reshape(1, 3 * D)
    for l in range(NUM_GGC):
        gm = _gather_e(msg, srcR).reshape(EPAD, D)[:E]
        m = jax.ops.segment_sum(gm, dst, num_segments=N)
        wn = Wggc[(l + 1) % NUM_GGC]
        h, msg = _tc_gru(m, h, wih, whh, bihr, bhhr, wn)

    return h


# speedup vs baseline: 1.2331x; 1.0011x over previous
import functools

import jax
import jax.numpy as jnp
from jax import lax
from jax.experimental import pallas as pl
from jax.experimental.pallas import tpu as pltpu
from jax.experimental.pallas import tpu_sc as plsc

N = 10000
E = 320000
D = 128
NUM_GCN = 30
NUM_GGC = 8

NC = 2
NS = 16
NW = NC * NS
CH = 128
EPW = -(-E // (NW * CH)) * CH
CHUNKS = EPW // CH
EPAD = EPW * NW

NPAD = 10016
RPS = NPAD // NS


def _make_agg(width):
    mesh = plsc.VectorSubcoreMesh(
        core_axis_name="c", subcore_axis_name="s", num_cores=NC, num_subcores=NS
    )

    @functools.partial(
        pl.kernel,
        out_type=jax.ShapeDtypeStruct((NC, NS, RPS, width), jnp.float32),
        mesh=mesh,
        scratch_types=[
            pltpu.VMEM((CHUNKS, CH), jnp.int32),
            pltpu.VMEM((CHUNKS, CH), jnp.int32),
            pltpu.VMEM((CH, width), jnp.float32),
            pltpu.VMEM_SHARED((NPAD, width), jnp.float32),
        ],
    )
    def agg(t_hbm, src_hbm, dst_hbm, z_hbm, out_hbm, src_v, dst_v, rows_v, acc):
        c = lax.axis_index("c")
        s = lax.axis_index("s")
        wid = c * NS + s
        pltpu.sync_copy(z_hbm, acc.at[pl.ds(s * RPS, RPS)])
        pltpu.sync_copy(src_hbm.at[wid], src_v)
        pltpu.sync_copy(dst_hbm.at[wid], dst_v)
        plsc.subcore_barrier()

        def body(j, carry):
            pltpu.sync_copy(t_hbm.at[src_v.at[j]], rows_v)
            pltpu.sync_copy(rows_v, acc.at[dst_v.at[j]], add=True)
            return carry

        lax.fori_loop(0, CHUNKS, body, 0)
        plsc.subcore_barrier()
        pltpu.sync_copy(acc.at[pl.ds(s * RPS, RPS)], out_hbm.at[c, s])

    return agg


_agg_d = _make_agg(D)


def _make_gather(chunks):
    mesh = plsc.VectorSubcoreMesh(
        core_axis_name="c", subcore_axis_name="s", num_cores=NC, num_subcores=NS
    )

    NB = 4

    @functools.partial(
        pl.kernel,
        out_type=jax.ShapeDtypeStruct((NW, chunks * CH, D), jnp.float32),
        mesh=mesh,
        scratch_types=[
            pltpu.VMEM((chunks, CH), jnp.int32),
            pltpu.VMEM((NB, CH, D), jnp.float32),
            pltpu.SemaphoreType.DMA((NB,)),
            pltpu.SemaphoreType.DMA((NB,)),
        ],
    )
    def gat(t_hbm, idx_hbm, out_hbm, idx_v, ring, gsem, wsem):
        c = lax.axis_index("c")
        s = lax.axis_index("s")
        wid = c * NS + s
        pltpu.sync_copy(idx_hbm.at[wid], idx_v)
        for k in range(NB - 1):
            if k < chunks:
                pltpu.async_copy(t_hbm.at[idx_v.at[k]], ring.at[k], gsem.at[k])

        def body(j, carry):
            par = lax.rem(j, NB)

            @pl.when(j >= 1)
            def _():
                pltpu.make_async_copy(
                    ring.at[lax.rem(j - 1, NB)],
                    out_hbm.at[wid, pl.ds((j - 1) * CH, CH)],
                    wsem.at[lax.rem(j - 1, NB)],
                ).wait()

            @pl.when(j + NB - 1 < chunks)
            def _():
                pltpu.async_copy(
                    t_hbm.at[idx_v.at[j + NB - 1]],
                    ring.at[lax.rem(j + NB - 1, NB)],
                    gsem.at[lax.rem(j + NB - 1, NB)])

            pltpu.make_async_copy(
                t_hbm.at[idx_v.at[j]], ring.at[par], gsem.at[par]).wait()
            pltpu.async_copy(
                ring.at[par], out_hbm.at[wid, pl.ds(j * CH, CH)], wsem.at[par])
            return carry

        lax.fori_loop(0, chunks, body, 0)
        last = lax.rem(jnp.int32(chunks - 1), NB)
        pltpu.make_async_copy(
            ring.at[last],
            out_hbm.at[wid, pl.ds((chunks - 1) * CH, CH)],
            wsem.at[last],
        ).wait()

    return gat


EA = E + N
EPW_A = -(-EA // (NW * CH)) * CH
CHUNKS_A = EPW_A // CH
EPAD_A = EPW_A * NW

_gather_e = _make_gather(CHUNKS)
_gather_a = _make_gather(CHUNKS_A)


BLK = 1000
GRID = N // BLK


def _tc_pre_body(x_r, w0_r, b0_r, wg0_r, c0_r, c1_r, dinv_o, t_o):
    deg = c0_r[:, 0:1] + c1_r[:, 0:1] + 1.0
    dinv = lax.rsqrt(deg)
    h = jnp.maximum(
        jnp.dot(x_r[...], w0_r[...], preferred_element_type=jnp.float32)
        + b0_r[...], 0.0)
    dinv_o[...] = jnp.broadcast_to(dinv, (BLK, D))
    t_o[...] = jnp.dot(h, wg0_r[...], preferred_element_type=jnp.float32)


def _tc_pre(x, w0, b0, wg0, c0, c1):
    return pl.pallas_call(
        _tc_pre_body,
        grid=(GRID,),
        in_specs=[
            pl.BlockSpec((BLK, 2 * D), lambda i: (i, 0)),
            pl.BlockSpec((2 * D, D), lambda i: (0, 0)),
            pl.BlockSpec((1, D), lambda i: (0, 0)),
            pl.BlockSpec((D, D), lambda i: (0, 0)),
            pl.BlockSpec((BLK, 16), lambda i: (i, 0)),
            pl.BlockSpec((BLK, 16), lambda i: (i, 0)),
        ],
        out_specs=[
            pl.BlockSpec((BLK, D), lambda i: (i, 0)),
            pl.BlockSpec((BLK, D), lambda i: (i, 0)),
        ],
        out_shape=[
            jax.ShapeDtypeStruct((N, D), jnp.float32),
            jax.ShapeDtypeStruct((N, D), jnp.float32),
        ],
    )(x, w0, b0, wg0, c0, c1)


def _tc_mm_body(m_r, bg_r, w_r, t_o):
    h = m_r[...] + bg_r[...]
    t_o[...] = jnp.dot(h, w_r[...], preferred_element_type=jnp.float32)


def _tc_mm(m, bg, w):
    return pl.pallas_call(
        _tc_mm_body,
        grid=(GRID,),
        in_specs=[
            pl.BlockSpec((BLK, D), lambda i: (i, 0)),
            pl.BlockSpec((1, D), lambda i: (0, 0)),
            pl.BlockSpec((D, D), lambda i: (0, 0)),
        ],
        out_specs=pl.BlockSpec((BLK, D), lambda i: (i, 0)),
        out_shape=jax.ShapeDtypeStruct((N, D), jnp.float32),
    )(m, bg, w)


def _tc_entry_body(m_r, bg_r, w_r, h_o, msg_o):
    h = m_r[...] + bg_r[...]
    h_o[...] = h
    msg_o[...] = jnp.dot(h, w_r[...], preferred_element_type=jnp.float32)


def _tc_entry(m, bg, w):
    return pl.pallas_call(
        _tc_entry_body,
        grid=(GRID,),
        in_specs=[
            pl.BlockSpec((BLK, D), lambda i: (i, 0)),
            pl.BlockSpec((1, D), lambda i: (0, 0)),
            pl.BlockSpec((D, D), lambda i: (0, 0)),
        ],
        out_specs=[
            pl.BlockSpec((BLK, D), lambda i: (i, 0)),
            pl.BlockSpec((BLK, D), lambda i: (i, 0)),
        ],
        out_shape=[
            jax.ShapeDtypeStruct((N, D), jnp.float32),
            jax.ShapeDtypeStruct((N, D), jnp.float32),
        ],
    )(m, bg, w)


def _tc_gru_body(m_r, h_r, wih_r, whh_r, bih_r, bhh_r, wn_r, h_o, msg_o):
    gi = jnp.dot(m_r[...], wih_r[...], preferred_element_type=jnp.float32) + bih_r[...]
    gh = jnp.dot(h_r[...], whh_r[...], preferred_element_type=jnp.float32) + bhh_r[...]
    h = h_r[...]
    r = jax.nn.sigmoid(gi[:, 0:D] + gh[:, 0:D])
    z = jax.nn.sigmoid(gi[:, D:2 * D] + gh[:, D:2 * D])
    n = jnp.tanh(gi[:, 2 * D:3 * D] + r * gh[:, 2 * D:3 * D])
    h2 = (1.0 - z) * n + z * h
    h_o[...] = h2
    msg_o[...] = jnp.dot(h2, wn_r[...], preferred_element_type=jnp.float32)


def _tc_gru(m, h, wih, whh, bih, bhh, wn):
    return pl.pallas_call(
        _tc_gru_body,
        grid=(GRID,),
        in_specs=[
            pl.BlockSpec((BLK, D), lambda i: (i, 0)),
            pl.BlockSpec((BLK, D), lambda i: (i, 0)),
            pl.BlockSpec((D, 3 * D), lambda i: (0, 0)),
            pl.BlockSpec((D, 3 * D), lambda i: (0, 0)),
            pl.BlockSpec((1, 3 * D), lambda i: (0, 0)),
            pl.BlockSpec((1, 3 * D), lambda i: (0, 0)),
            pl.BlockSpec((D, D), lambda i: (0, 0)),
        ],
        out_specs=[
            pl.BlockSpec((BLK, D), lambda i: (i, 0)),
            pl.BlockSpec((BLK, D), lambda i: (i, 0)),
        ],
        out_shape=[
            jax.ShapeDtypeStruct((N, D), jnp.float32),
            jax.ShapeDtypeStruct((N, D), jnp.float32),
        ],
    )(m, h, wih, whh, bih, bhh, wn)


def kernel(x, edge_index, W0, b0, Wg, bg, Wggc, W_ih, W_hh, b_ih, b_hh):
    src = edge_index[0]
    dst = edge_index[1]
    padn = EPAD - E
    srcp = jnp.concatenate([src, jnp.zeros((padn,), jnp.int32)])
    dstp = jnp.concatenate([dst, jnp.full((padn,), N, jnp.int32)])
    srcR = srcp.reshape(NW, CHUNKS, CH)
    dstR = dstp.reshape(NW, CHUNKS, CH)

    zD = jnp.zeros((RPS, D), jnp.float32)
    onesD = jnp.ones((N, D), jnp.float32)

    cnt = _agg_d(onesD, srcR, dstR, zD).reshape(NC, NPAD, D)
    c0 = cnt[0, :N, :16]
    c1 = cnt[1, :N, :16]

    b0r = b0.reshape(1, D)
    dinv_b, t = _tc_pre(x, W0, b0r, Wg[0], c0, c1)
    dinv = dinv_b[:, 0]

    loop = jnp.arange(N, dtype=src.dtype)
    src_a = jnp.concatenate([src, loop])
    dst_a = jnp.concatenate([dst, loop])
    norm = dinv[src_a] * dinv[dst_a]

    pad_a = EPAD_A - EA
    srcaR = jnp.concatenate([src_a, jnp.zeros((pad_a,), jnp.int32)]).reshape(
        NW, CHUNKS_A, CH)

    def agg_gcn(t):
        g = _gather_a(t, srcaR).reshape(EPAD_A, D)[:EA]
        return jax.ops.segment_sum(g * norm[:, None], dst_a, num_segments=N)

    for i in range(NUM_GCN - 1):
        m = agg_gcn(t)
        t = _tc_mm(m, bg[i].reshape(1, D), Wg[i + 1])

    m = agg_gcn(t)
    h, msg = _tc_entry(m, bg[NUM_GCN - 1].reshape(1, D), Wggc[0])

    wih = W_ih.T
    whh = W_hh.T
    bihr = b_ih.reshape(1, 3 * D)
    bhhr = b_hh.reshape(1, 3 * D)
    for l in range(NUM_GGC):
        gm = _gather_e(msg, srcR).reshape(EPAD, D)[:E]
        m = jax.ops.segment_sum(gm, dst, num_segments=N)
        wn = Wggc[(l + 1) % NUM_GGC]
        h, msg = _tc_gru(m, h, wih, whh, bihr, bhhr, wn)

    return h
